# initial kernel scaffold (unmeasured)
import jax
import jax.numpy as jnp
from jax import lax
from jax.experimental import pallas as pl
from jax.experimental.pallas import tpu as pltpu

S = 1024
D = 2048
DC = 128
H = 16
DH = 128
DR = 32


def _dot(a, b):
    return lax.dot_general(
        a, b, (((1,), (0,)), ((), ())), preferred_element_type=jnp.float32)


def _dot_nt(a, b):
    return lax.dot_general(
        a, b, (((1,), (1,)), ((), ())), preferred_element_type=jnp.float32)


def kernel(x, Wdkv, Wuk, Wuv, Wq, Wqr, Wkr, Wo):
    x2 = x.reshape(S, D)

    def body_a(x_ref, wdkv_ref, wuk_ref, wuv_ref, wq_ref, wqr_ref, wkr_ref,
               k_ref, v_ref, q_ref, qr_ref, kr_ref,
               c_ref, c_rx_ref, wuk_rx_ref, wuv_rx_ref, send_sems, recv_sems):
        my_x = lax.axis_index("x")
        my_y = lax.axis_index("y")
        my_z = lax.axis_index("z")
        nbr = (1 - my_x, my_y, my_z)

        barrier = pltpu.get_barrier_semaphore()
        pl.semaphore_signal(barrier, inc=1, device_id=nbr,
                            device_id_type=pl.DeviceIdType.MESH)
        pl.semaphore_wait(barrier, 1)

        c_ref[...] = _dot(x_ref[...], wdkv_ref[...])

        rdmas = []
        for i, (src, dst) in enumerate([(c_ref, c_rx_ref),
                                        (wuk_ref, wuk_rx_ref),
                                        (wuv_ref, wuv_rx_ref)]):
            r = pltpu.make_async_remote_copy(
                src_ref=src, dst_ref=dst,
                send_sem=send_sems.at[i], recv_sem=recv_sems.at[i],
                device_id=nbr, device_id_type=pl.DeviceIdType.MESH)
            r.start()
            rdmas.append(r)

        q_ref[...] = _dot(x_ref[...], wq_ref[...])
        qr_ref[...] = _dot(x_ref[...], wqr_ref[...])
        kr_ref[...] = _dot(x_ref[...], wkr_ref[...])

        for r in rdmas:
            r.wait()

        k_ref[...] = (_dot(c_ref[...], wuk_ref[...])
                      + _dot(c_rx_ref[...], wuk_rx_ref[...]))
        v_ref[...] = (_dot(c_ref[...], wuv_ref[...])
                      + _dot(c_rx_ref[...], wuv_rx_ref[...]))

    k, v, q, qr, kr = pl.pallas_call(
        body_a,
        out_shape=[
            jax.ShapeDtypeStruct((S, D), jnp.float32),
            jax.ShapeDtypeStruct((S, D), jnp.float32),
            jax.ShapeDtypeStruct((S, D), jnp.float32),
            jax.ShapeDtypeStruct((S, H * DR), jnp.float32),
            jax.ShapeDtypeStruct((S, DR), jnp.float32),
        ],
        in_specs=[pl.BlockSpec(memory_space=pltpu.VMEM)] * 7,
        out_specs=[pl.BlockSpec(memory_space=pltpu.VMEM)] * 5,
        scratch_shapes=[
            pltpu.VMEM((S, DC), jnp.float32),
            pltpu.VMEM((S, DC), jnp.float32),
            pltpu.VMEM((DC, D), jnp.float32),
            pltpu.VMEM((DC, D), jnp.float32),
            pltpu.SemaphoreType.DMA((3,)),
            pltpu.SemaphoreType.DMA((3,)),
        ],
        compiler_params=pltpu.CompilerParams(collective_id=0),
    )(x2, Wdkv, Wuk, Wuv, Wq, Wqr, Wkr)

    def body_b(q_ref, qr_ref, kr_ref, k_ref, v_ref, wo_ref, out_ref):
        scale = (DH + DR) ** -0.5
        kr_all = kr_ref[...]
        out_ref[...] = jnp.zeros((S, D), jnp.float32)
        for h in range(H):
            qh = q_ref[:, h * DH:(h + 1) * DH]
            kh = k_ref[:, h * DH:(h + 1) * DH]
            qrh = qr_ref[:, h * DR:(h + 1) * DR]
            s = (_dot_nt(qh, kh) + _dot_nt(qrh, kr_all)) * scale
            m = jnp.max(s, axis=1, keepdims=True)
            e = jnp.exp(s - m)
            p = e / jnp.sum(e, axis=1, keepdims=True)
            oh = _dot(p, v_ref[:, h * DH:(h + 1) * DH])
            out_ref[...] += _dot(oh, wo_ref[h * DH:(h + 1) * DH, :])

    out = pl.pallas_call(
        body_b,
        out_shape=jax.ShapeDtypeStruct((S, D), jnp.float32),
        in_specs=[pl.BlockSpec(memory_space=pltpu.VMEM)] * 6,
        out_specs=pl.BlockSpec(memory_space=pltpu.VMEM),
    )(q, qr, kr, k, v, Wo)

    return out.reshape(1, S, D)


# baseline (device time: 136650 ns/iter reference)
import jax
import jax.numpy as jnp
from jax import lax
from jax.experimental import pallas as pl
from jax.experimental.pallas import tpu as pltpu

S = 1024
D = 2048
DC = 128
H = 16
DH = 128
DR = 32

_VMEM = pl.BlockSpec(memory_space=pltpu.VMEM)


def _dot(a, b):
    return lax.dot_general(
        a, b, (((1,), (0,)), ((), ())), preferred_element_type=jnp.float32)


def _dot_nt(a, b):
    return lax.dot_general(
        a, b, (((1,), (1,)), ((), ())), preferred_element_type=jnp.float32)


def kernel(x, Wdkv, Wuk, Wuv, Wq, Wqr, Wkr, Wo):
    x2 = x.reshape(S, D)

    def body_a(x_ref, wdkv_ref, wuk_ref, wuv_ref, wqr_ref, wkr_ref,
               k_ref, v_ref, qr_ref, kr_ref,
               c_ref, c_rx_ref, wuk_rx_ref, wuv_rx_ref, send_sems, recv_sems):
        my_x = lax.axis_index("x")
        my_y = lax.axis_index("y")
        my_z = lax.axis_index("z")
        nbr = (1 - my_x, my_y, my_z)

        barrier = pltpu.get_barrier_semaphore()
        pl.semaphore_signal(barrier, inc=1, device_id=nbr,
                            device_id_type=pl.DeviceIdType.MESH)
        pl.semaphore_wait(barrier, 1)

        c_ref[...] = _dot(x_ref[...], wdkv_ref[...])

        rdmas = []
        for i, (src, dst) in enumerate([(c_ref, c_rx_ref),
                                        (wuk_ref, wuk_rx_ref),
                                        (wuv_ref, wuv_rx_ref)]):
            r = pltpu.make_async_remote_copy(
                src_ref=src, dst_ref=dst,
                send_sem=send_sems.at[i], recv_sem=recv_sems.at[i],
                device_id=nbr, device_id_type=pl.DeviceIdType.MESH)
            r.start()
            rdmas.append(r)

        qr_ref[...] = _dot(x_ref[...], wqr_ref[...])
        kr_ref[...] = _dot(x_ref[...], wkr_ref[...])

        for r in rdmas:
            r.wait()

        k_ref[...] = (_dot(c_ref[...], wuk_ref[...])
                      + _dot(c_rx_ref[...], wuk_rx_ref[...]))
        v_ref[...] = (_dot(c_ref[...], wuv_ref[...])
                      + _dot(c_rx_ref[...], wuv_rx_ref[...]))

    k, v, qr, kr = pl.pallas_call(
        body_a,
        out_shape=[
            jax.ShapeDtypeStruct((S, D), jnp.float32),
            jax.ShapeDtypeStruct((S, D), jnp.float32),
            jax.ShapeDtypeStruct((S, H * DR), jnp.float32),
            jax.ShapeDtypeStruct((S, DR), jnp.float32),
        ],
        in_specs=[_VMEM] * 6,
        out_specs=[_VMEM] * 4,
        scratch_shapes=[
            pltpu.VMEM((S, DC), jnp.float32),
            pltpu.VMEM((S, DC), jnp.float32),
            pltpu.VMEM((DC, D), jnp.float32),
            pltpu.VMEM((DC, D), jnp.float32),
            pltpu.SemaphoreType.DMA((3,)),
            pltpu.SemaphoreType.DMA((3,)),
        ],
        compiler_params=pltpu.CompilerParams(collective_id=0),
    )(x2, Wdkv, Wuk, Wuv, Wqr, Wkr)

    def body_b(x_ref, wq_ref, q_ref):
        q_ref[...] = _dot(x_ref[...], wq_ref[...])

    q = pl.pallas_call(
        body_b,
        out_shape=jax.ShapeDtypeStruct((S, D), jnp.float32),
        in_specs=[_VMEM] * 2,
        out_specs=_VMEM,
    )(x2, Wq)

    def body_c(q_ref, qr_ref, kr_ref, k_ref, v_ref, o_ref):
        scale = (DH + DR) ** -0.5
        kr_all = kr_ref[...]
        for h in range(H):
            qh = q_ref[:, h * DH:(h + 1) * DH]
            kh = k_ref[:, h * DH:(h + 1) * DH]
            qrh = qr_ref[:, h * DR:(h + 1) * DR]
            s = (_dot_nt(qh, kh) + _dot_nt(qrh, kr_all)) * scale
            m = jnp.max(s, axis=1, keepdims=True)
            e = jnp.exp(s - m)
            p = e / jnp.sum(e, axis=1, keepdims=True)
            o_ref[:, h * DH:(h + 1) * DH] = _dot(p, v_ref[:, h * DH:(h + 1) * DH])

    o = pl.pallas_call(
        body_c,
        out_shape=jax.ShapeDtypeStruct((S, D), jnp.float32),
        in_specs=[_VMEM] * 5,
        out_specs=_VMEM,
    )(q, qr, kr, k, v)

    def body_d(o_ref, wo_ref, out_ref):
        out_ref[...] = _dot(o_ref[...], wo_ref[...])

    out = pl.pallas_call(
        body_d,
        out_shape=jax.ShapeDtypeStruct((S, D), jnp.float32),
        in_specs=[_VMEM] * 2,
        out_specs=_VMEM,
    )(o, Wo)

    return out.reshape(1, S, D)
